# baseline (speedup 1.0000x reference)
import jax
import jax.numpy as jnp
from jax.experimental import pallas as pl
from jax.experimental.pallas import tpu as pltpu

H = 2048
E = 8
I = 4096
NH = 32
NKV = 8
HD = 64
T = 64
EPS = 1e-05
THETA = 1000000.0
HALF = HD // 2
NEG = -1e30

IBLK = 512
W2BLK = 1024
NT = I // IBLK


def _softmax_rows(s):
    m = jnp.max(s, axis=1, keepdims=True)
    p = jnp.exp(s - m)
    return p / jnp.sum(p, axis=1, keepdims=True)


def _fused_kernel(cs_ref, hs_ref, wqkv_ref, wo_ref, ln1w_ref, ln1b_ref,
                  gate_ref, ln2w_ref, ln2b_ref, w1_ref, w3_ref, w2_ref,
                  out_ref, ctx_ref, x_ref, wc_ref):
    e = pl.program_id(0)
    t = pl.program_id(1)

    @pl.when((e == 0) & (t == 0))
    def _prologue():
        hs = hs_ref[...]
        mu = jnp.mean(hs, axis=1, keepdims=True)
        var = jnp.mean((hs - mu) ** 2, axis=1, keepdims=True)
        xa = (hs - mu) / jnp.sqrt(var + EPS) * ln1w_ref[...] + ln1b_ref[...]
        qkv = jnp.dot(xa.astype(jnp.bfloat16), wqkv_ref[...],
                      preferred_element_type=jnp.float32)

        cos = cs_ref[:, :HALF]
        sin = cs_ref[:, HALF:]

        def rope(xh):
            x1 = xh[:, :HALF]
            x2 = xh[:, HALF:]
            return jnp.concatenate(
                [x1 * cos - x2 * sin, x2 * cos + x1 * sin], axis=1)

        row = jax.lax.broadcasted_iota(jnp.int32, (T, T), 0)
        col = jax.lax.broadcasted_iota(jnp.int32, (T, T), 1)
        causal = row >= col
        inv_sqrt_hd = 1.0 / jnp.sqrt(float(HD))

        k_base = NH * HD
        v_base = (NH + NKV) * HD
        for g in range(NKV):
            kg = rope(qkv[:, k_base + g * HD:k_base + (g + 1) * HD])
            vg = qkv[:, v_base + g * HD:v_base + (g + 1) * HD]
            for r in range(NH // NKV):
                hidx = g * (NH // NKV) + r
                qh = rope(qkv[:, hidx * HD:(hidx + 1) * HD])
                s = jax.lax.dot_general(qh, kg, (((1,), (1,)), ((), ())))
                s = s * inv_sqrt_hd
                s = jnp.where(causal, s, NEG)
                p = _softmax_rows(s)
                ctx_ref[:, hidx * HD:(hidx + 1) * HD] = (
                    jnp.dot(p, vg).astype(jnp.bfloat16))

        hv = hs + jnp.dot(ctx_ref[...].astype(jnp.float32), wo_ref[...])

        mu2 = jnp.mean(hv, axis=1, keepdims=True)
        var2 = jnp.mean((hv - mu2) ** 2, axis=1, keepdims=True)
        x = ((hv - mu2) / jnp.sqrt(var2 + EPS) * ln2w_ref[...]
             + ln2b_ref[...])
        x_ref[...] = x
        scores = jnp.dot(x, gate_ref[...])

        eidx = jax.lax.broadcasted_iota(jnp.int32, (T, E), 1)
        jit2 = 2 * 0.01

        mlt = jnp.max(scores, axis=1, keepdims=True)
        amax1 = jnp.min(jnp.where(scores == mlt, eidx, E), axis=1,
                        keepdims=True)
        factor = jnp.maximum(jnp.abs(scores), mlt)
        mask = (mlt - scores) / factor > jit2
        mg = jnp.where(mask, NEG, scores)
        p1 = _softmax_rows(mg)
        m1 = jnp.sum(jnp.where(eidx == amax1, p1, 0.0), axis=1,
                     keepdims=True)

        masked = jnp.where(eidx == amax1, NEG, scores)
        mlt2 = jnp.max(masked, axis=1, keepdims=True)
        amax2 = jnp.min(jnp.where(masked == mlt2, eidx, E), axis=1,
                        keepdims=True)
        factor2 = jnp.maximum(jnp.abs(scores), mlt2)
        mask2 = (mlt2 - scores) / factor2 > jit2
        mg2 = jnp.where(mask2, NEG, masked)
        p2 = _softmax_rows(mg2)
        m2 = jnp.sum(jnp.where(eidx == amax2, p2, 0.0), axis=1,
                     keepdims=True)

        wc_ref[...] = (jnp.where(eidx == amax1, m1, 0.0)
                       + jnp.where(eidx == amax2, m2, 0.0))
        out_ref[...] = hv

    x = x_ref[...]
    lane = jax.lax.broadcasted_iota(jnp.int32, (T, E), 1)
    wsel = jnp.sum(jnp.where(lane == e, wc_ref[...], 0.0), axis=1,
                   keepdims=True)

    w1b = w1_ref[0]
    w3b = w3_ref[0]
    half = jax.lax.rem(t, W2BLK // IBLK)
    w2b = w2_ref[0, :, pl.dslice(half * IBLK, IBLK)]
    a1 = jax.lax.dot_general(x, w1b, (((1,), (1,)), ((), ())))
    a3 = jax.lax.dot_general(x, w3b, (((1,), (1,)), ((), ())))
    gact = a1 * jax.lax.logistic(a1) * a3
    y = jax.lax.dot_general(gact, w2b, (((1,), (1,)), ((), ())))
    out_ref[...] += wsel * y


def kernel(positions, hidden_states, Wqkv, Wo, gate_W, w1, w2, w3,
           ln1_w, ln1_b, ln2_w, ln2_b):
    inv = 1.0 / (THETA ** (jnp.arange(HALF, dtype=jnp.float32) / HALF))
    ang = positions.astype(jnp.float32)[:, None] * inv[None, :]
    cs = jnp.concatenate([jnp.cos(ang), jnp.sin(ang)], axis=1)

    out = pl.pallas_call(
        _fused_kernel,
        grid=(E, NT),
        in_specs=[
            pl.BlockSpec((T, HD), lambda e, t: (0, 0)),
            pl.BlockSpec((T, H), lambda e, t: (0, 0)),
            pl.BlockSpec((H, (NH + 2 * NKV) * HD), lambda e, t: (0, 0)),
            pl.BlockSpec((NH * HD, H), lambda e, t: (0, 0)),
            pl.BlockSpec((1, H), lambda e, t: (0, 0)),
            pl.BlockSpec((1, H), lambda e, t: (0, 0)),
            pl.BlockSpec((H, E), lambda e, t: (0, 0)),
            pl.BlockSpec((1, H), lambda e, t: (0, 0)),
            pl.BlockSpec((1, H), lambda e, t: (0, 0)),
            pl.BlockSpec((1, IBLK, H), lambda e, t: (e, t, 0)),
            pl.BlockSpec((1, IBLK, H), lambda e, t: (e, t, 0)),
            pl.BlockSpec((1, H, W2BLK),
                         lambda e, t: (e, 0, t // (W2BLK // IBLK))),
        ],
        out_specs=pl.BlockSpec((T, H), lambda e, t: (0, 0)),
        out_shape=jax.ShapeDtypeStruct((T, H), jnp.float32),
        scratch_shapes=[
            pltpu.VMEM((T, NH * HD), jnp.bfloat16),
            pltpu.VMEM((T, H), jnp.float32),
            pltpu.VMEM((T, E), jnp.float32),
        ],
        compiler_params=pltpu.CompilerParams(
            dimension_semantics=("arbitrary", "arbitrary"),
            vmem_limit_bytes=110 * 1024 * 1024,
        ),
    )(cs, hidden_states, Wqkv.astype(jnp.bfloat16), Wo,
      ln1_w.reshape(1, H), ln1_b.reshape(1, H),
      gate_W, ln2_w.reshape(1, H), ln2_b.reshape(1, H), w1, w3, w2)
    return out


# speedup vs baseline: 1.1423x; 1.1423x over previous
import jax
import jax.numpy as jnp
from jax.experimental import pallas as pl
from jax.experimental.pallas import tpu as pltpu

H = 2048
E = 8
I = 4096
NH = 32
NKV = 8
HD = 64
T = 64
EPS = 1e-05
THETA = 1000000.0
HALF = HD // 2
NEG = -1e30

IBLK = 512
NT = I // IBLK


def _softmax_rows(s):
    m = jnp.max(s, axis=1, keepdims=True)
    p = jnp.exp(s - m)
    return p / jnp.sum(p, axis=1, keepdims=True)


def _attn_kernel(cs_ref, hs_ref, wqkv_ref, wo_ref, ln1w_ref, ln1b_ref,
                 h_ref, ctx_ref):
    hs = hs_ref[...]
    mu = jnp.mean(hs, axis=1, keepdims=True)
    var = jnp.mean((hs - mu) ** 2, axis=1, keepdims=True)
    x = (hs - mu) / jnp.sqrt(var + EPS) * ln1w_ref[...] + ln1b_ref[...]
    qkv = jnp.dot(x, wqkv_ref[...])

    cos = cs_ref[:, :HALF]
    sin = cs_ref[:, HALF:]

    def rope(xh):
        x1 = xh[:, :HALF]
        x2 = xh[:, HALF:]
        return jnp.concatenate([x1 * cos - x2 * sin, x2 * cos + x1 * sin],
                               axis=1)

    row = jax.lax.broadcasted_iota(jnp.int32, (T, T), 0)
    col = jax.lax.broadcasted_iota(jnp.int32, (T, T), 1)
    causal = row >= col
    inv_sqrt_hd = 1.0 / jnp.sqrt(float(HD))

    k_base = NH * HD
    v_base = (NH + NKV) * HD
    for g in range(NKV):
        kg = rope(qkv[:, k_base + g * HD:k_base + (g + 1) * HD])
        vg = qkv[:, v_base + g * HD:v_base + (g + 1) * HD]
        for r in range(NH // NKV):
            hidx = g * (NH // NKV) + r
            qh = rope(qkv[:, hidx * HD:(hidx + 1) * HD])
            s = jax.lax.dot_general(qh, kg, (((1,), (1,)), ((), ())))
            s = s * inv_sqrt_hd
            s = jnp.where(causal, s, NEG)
            p = _softmax_rows(s)
            ctx_ref[:, hidx * HD:(hidx + 1) * HD] = jnp.dot(p, vg)

    attn = jnp.dot(ctx_ref[...], wo_ref[...])
    h_ref[...] = hs + attn


def _moe_kernel(h_ref, gate_ref, ln2w_ref, ln2b_ref, w1_ref, w3_ref, w2_ref,
                out_ref, x_ref, wc_ref):
    e = pl.program_id(0)
    t = pl.program_id(1)

    @pl.when((e == 0) & (t == 0))
    def _prologue():
        hv = h_ref[...]
        mu = jnp.mean(hv, axis=1, keepdims=True)
        var = jnp.mean((hv - mu) ** 2, axis=1, keepdims=True)
        x = (hv - mu) / jnp.sqrt(var + EPS) * ln2w_ref[...] + ln2b_ref[...]
        x_ref[...] = x
        scores = jnp.dot(x, gate_ref[...])

        eidx = jax.lax.broadcasted_iota(jnp.int32, (T, E), 1)
        jit2 = 2 * 0.01

        mlt = jnp.max(scores, axis=1, keepdims=True)
        amax1 = jnp.min(jnp.where(scores == mlt, eidx, E), axis=1,
                        keepdims=True)
        factor = jnp.maximum(jnp.abs(scores), mlt)
        mask = (mlt - scores) / factor > jit2
        mg = jnp.where(mask, NEG, scores)
        p1 = _softmax_rows(mg)
        m1 = jnp.sum(jnp.where(eidx == amax1, p1, 0.0), axis=1, keepdims=True)

        masked = jnp.where(eidx == amax1, NEG, scores)
        mlt2 = jnp.max(masked, axis=1, keepdims=True)
        amax2 = jnp.min(jnp.where(masked == mlt2, eidx, E), axis=1,
                        keepdims=True)
        factor2 = jnp.maximum(jnp.abs(scores), mlt2)
        mask2 = (mlt2 - scores) / factor2 > jit2
        mg2 = jnp.where(mask2, NEG, masked)
        p2 = _softmax_rows(mg2)
        m2 = jnp.sum(jnp.where(eidx == amax2, p2, 0.0), axis=1, keepdims=True)

        wc_ref[...] = (jnp.where(eidx == amax1, m1, 0.0)
                       + jnp.where(eidx == amax2, m2, 0.0))
        out_ref[...] = hv

    x = x_ref[...]
    lane = jax.lax.broadcasted_iota(jnp.int32, (T, E), 1)
    wsel = jnp.sum(jnp.where(lane == e, wc_ref[...], 0.0), axis=1,
                   keepdims=True)

    w1b = w1_ref[0]
    w3b = w3_ref[0]
    w2b = w2_ref[0]
    a1 = jax.lax.dot_general(x, w1b, (((1,), (1,)), ((), ())))
    a3 = jax.lax.dot_general(x, w3b, (((1,), (1,)), ((), ())))
    gact = a1 * jax.lax.logistic(a1) * a3
    y = jax.lax.dot_general(gact, w2b, (((1,), (1,)), ((), ())))
    out_ref[...] += wsel * y


def kernel(positions, hidden_states, Wqkv, Wo, gate_W, w1, w2, w3,
           ln1_w, ln1_b, ln2_w, ln2_b):
    inv = 1.0 / (THETA ** (jnp.arange(HALF, dtype=jnp.float32) / HALF))
    ang = positions.astype(jnp.float32)[:, None] * inv[None, :]
    cs = jnp.concatenate([jnp.cos(ang), jnp.sin(ang)], axis=1)

    h = pl.pallas_call(
        _attn_kernel,
        out_shape=jax.ShapeDtypeStruct((T, H), jnp.float32),
        scratch_shapes=[pltpu.VMEM((T, NH * HD), jnp.float32)],
    )(cs, hidden_states, Wqkv, Wo, ln1_w.reshape(1, H), ln1_b.reshape(1, H))

    out = pl.pallas_call(
        _moe_kernel,
        grid=(E, NT),
        in_specs=[
            pl.BlockSpec((T, H), lambda e, t: (0, 0)),
            pl.BlockSpec((H, E), lambda e, t: (0, 0)),
            pl.BlockSpec((1, H), lambda e, t: (0, 0)),
            pl.BlockSpec((1, H), lambda e, t: (0, 0)),
            pl.BlockSpec((1, IBLK, H), lambda e, t: (e, t, 0)),
            pl.BlockSpec((1, IBLK, H), lambda e, t: (e, t, 0)),
            pl.BlockSpec((1, H, IBLK), lambda e, t: (e, 0, t)),
        ],
        out_specs=pl.BlockSpec((T, H), lambda e, t: (0, 0)),
        out_shape=jax.ShapeDtypeStruct((T, H), jnp.float32),
        scratch_shapes=[
            pltpu.VMEM((T, H), jnp.float32),
            pltpu.VMEM((T, E), jnp.float32),
        ],
        compiler_params=pltpu.CompilerParams(
            dimension_semantics=("arbitrary", "arbitrary"),
        ),
    )(h, gate_W, ln2_w.reshape(1, H), ln2_b.reshape(1, H), w1, w3, w2)
    return out
